# baseline (speedup 1.0000x reference)
import jax
import jax.numpy as jnp
from jax import lax
from jax.experimental import pallas as pl
from jax.experimental.pallas import tpu as pltpu
from jax.experimental.pallas import tpu_sc as plsc

N = 10000
F = 128
N_PAD = 10112
NS = 16
NC = 2
CHUNK = 128
CHUNKS_PER_TILE = 80
E_PAD = NS * CHUNKS_PER_TILE * CHUNK
EPT = E_PAD // NS
ROWS_PER_TILE = N_PAD // NS
BM = 400


def _mm_body(xu_ref, xi_ref, wu_ref, wi_ref,
             tabu_ref, ru_ref, tabi_ref, ri_ref):
    zu = jnp.dot(xu_ref[...], wu_ref[...], preferred_element_type=jnp.float32)
    tabu_ref[...] = zu[:, :F]
    ru_ref[...] = zu[:, F:]
    zi = jnp.dot(xi_ref[...], wi_ref[...], preferred_element_type=jnp.float32)
    tabi_ref[...] = zi[:, :F]
    ri_ref[...] = zi[:, F:]


def _mm(x_u, x_i, w_u, w_i):
    k = x_u.shape[1]
    return pl.pallas_call(
        _mm_body,
        grid=(N // BM,),
        in_specs=[
            pl.BlockSpec((BM, k), lambda i: (i, 0)),
            pl.BlockSpec((BM, k), lambda i: (i, 0)),
            pl.BlockSpec((k, 2 * F), lambda i: (0, 0)),
            pl.BlockSpec((k, 2 * F), lambda i: (0, 0)),
        ],
        out_specs=[
            pl.BlockSpec((BM, F), lambda i: (i, 0)),
            pl.BlockSpec((BM, F), lambda i: (i, 0)),
            pl.BlockSpec((BM, F), lambda i: (i, 0)),
            pl.BlockSpec((BM, F), lambda i: (i, 0)),
        ],
        out_shape=[
            jax.ShapeDtypeStruct((N, F), jnp.float32),
            jax.ShapeDtypeStruct((N, F), jnp.float32),
            jax.ShapeDtypeStruct((N, F), jnp.float32),
            jax.ShapeDtypeStruct((N, F), jnp.float32),
        ],
    )(x_u, x_i, w_u, w_i)


def _epi_body_relu(au, cu, ru, bu, ai, ci, ri, bi, ou, oi):
    _epi_common(au, cu, ru, bu, ai, ci, ri, bi, ou, oi, True)


def _epi_body_lin(au, cu, ru, bu, ai, ci, ri, bi, ou, oi):
    _epi_common(au, cu, ru, bu, ai, ci, ri, bi, ou, oi, False)


def _epi_common(au, cu, ru, bu, ai, ci, ri, bi, ou, oi, relu):
    def one(acc_ref, cnt_ref, r_ref, b_ref, o_ref):
        cnt = jnp.maximum(cnt_ref[...], 1.0)
        out = acc_ref[...] / cnt + b_ref[...] + r_ref[...]
        if relu:
            out = jnp.maximum(out, 0.0)
        o_ref[...] = out
    one(au, cu, ru, bu, ou)
    one(ai, ci, ri, bi, oi)


def _epi(acc_u, cnt_u, r_u, b_u, acc_i, cnt_i, r_i, b_i, relu):
    body = _epi_body_relu if relu else _epi_body_lin
    blk = pl.BlockSpec((BM, F), lambda i: (i, 0))
    bias = pl.BlockSpec((1, F), lambda i: (0, 0))
    return pl.pallas_call(
        body,
        grid=(N // BM,),
        in_specs=[blk, blk, blk, bias, blk, blk, blk, bias],
        out_specs=[blk, blk],
        out_shape=[
            jax.ShapeDtypeStruct((N, F), jnp.float32),
            jax.ShapeDtypeStruct((N, F), jnp.float32),
        ],
    )(acc_u, cnt_u, r_u, b_u.reshape(1, F),
      acc_i, cnt_i, r_i, b_i.reshape(1, F))


CH_S = 128
NBUF = 2
KCH = 10
OUTER = EPT // CH_S // KCH
NBUF_C = 10
OUTER_C = CHUNKS_PER_TILE // NBUF_C


def _sc_scatter_body(tab_u, tab_i, s_ui, d_ui, s_iu, d_iu, zinit,
                     out_i, out_u, src_b, dst_b, rows_b, acc_sh,
                     sem_i, sem_g, sem_s):
    cid = lax.axis_index("c")
    sid = lax.axis_index("s")
    row0 = sid * ROWS_PER_TILE
    pltpu.sync_copy(zinit, acc_sh.at[pl.ds(row0, ROWS_PER_TILE)])
    plsc.subcore_barrier()

    def run(tab, s_hbm, d_hbm):
        def issue_idx(g0):
            for j in range(KCH):
                base = sid * EPT + (g0 * KCH + j) * CH_S
                pltpu.async_copy(s_hbm.at[pl.ds(base, CH_S)],
                                 src_b[j], sem_i[j])
                pltpu.async_copy(d_hbm.at[pl.ds(base, CH_S)],
                                 dst_b[j], sem_i[j])

        def wait_idx(g0, j, which):
            base = sid * EPT + (g0 * KCH + j) * CH_S
            ref = s_hbm if which == 0 else d_hbm
            buf = src_b[j] if which == 0 else dst_b[j]
            pltpu.make_async_copy(ref.at[pl.ds(base, CH_S)],
                                  buf, sem_i[j]).wait()

        issue_idx(0)

        def outer(g0, carry):
            gs = [None] * KCH
            ss = [None] * KCH
            for j in range(KCH):
                slot = j % NBUF
                if j >= NBUF:
                    ss[j - NBUF].wait()
                wait_idx(g0, j, 0)
                gs[j] = pltpu.async_copy(tab.at[src_b[j]],
                                         rows_b[slot], sem_g[slot])
                if j >= 1:
                    gs[j - 1].wait()
                    wait_idx(g0, j - 1, 1)
                    ss[j - 1] = pltpu.async_copy(
                        rows_b[(j - 1) % NBUF],
                        acc_sh.at[dst_b[j - 1]], sem_s[(j - 1) % NBUF],
                        add=True)
            gs[KCH - 1].wait()
            wait_idx(g0, KCH - 1, 1)
            ss[KCH - 1] = pltpu.async_copy(
                rows_b[(KCH - 1) % NBUF],
                acc_sh.at[dst_b[KCH - 1]], sem_s[(KCH - 1) % NBUF],
                add=True)

            ss[KCH - 2].wait()
            ss[KCH - 1].wait()

            @pl.when(g0 + 1 < OUTER)
            def _():
                issue_idx(g0 + 1)
            return carry
        lax.fori_loop(0, OUTER, outer, 0)

    @pl.when(cid == 0)
    def _():
        run(tab_u, s_ui, d_ui)

    @pl.when(cid == 1)
    def _():
        run(tab_i, s_iu, d_iu)

    plsc.subcore_barrier()

    @pl.when(cid == 0)
    def _():
        pltpu.sync_copy(acc_sh.at[pl.ds(row0, ROWS_PER_TILE)],
                        out_i.at[pl.ds(row0, ROWS_PER_TILE)])

    @pl.when(cid == 1)
    def _():
        pltpu.sync_copy(acc_sh.at[pl.ds(row0, ROWS_PER_TILE)],
                        out_u.at[pl.ds(row0, ROWS_PER_TILE)])


def _sc_scatter(tab_u, tab_i, s_ui, d_ui, s_iu, d_iu, zinit):
    mesh = plsc.VectorSubcoreMesh(core_axis_name="c", subcore_axis_name="s",
                                  num_cores=NC, num_subcores=NS)
    f = pl.kernel(
        _sc_scatter_body,
        out_type=(jax.ShapeDtypeStruct((N_PAD, F), jnp.float32),
                  jax.ShapeDtypeStruct((N_PAD, F), jnp.float32)),
        mesh=mesh,
        scratch_types=[
            [pltpu.VMEM((CH_S,), jnp.int32) for _ in range(KCH)],
            [pltpu.VMEM((CH_S,), jnp.int32) for _ in range(KCH)],
            [pltpu.VMEM((CH_S, F), jnp.float32) for _ in range(NBUF)],
            pltpu.VMEM_SHARED((N_PAD, F), jnp.float32),
            [pltpu.SemaphoreType.DMA for _ in range(KCH)],
            [pltpu.SemaphoreType.DMA for _ in range(NBUF)],
            [pltpu.SemaphoreType.DMA for _ in range(NBUF)],
        ],
    )
    return f(tab_u, tab_i, s_ui, d_ui, s_iu, d_iu, zinit)


def _sc_count_body(d_ui, d_iu, ones_rows, zinit, out_i, out_u,
                   dst_b, rows_v, acc_sh, sem_i, sem_s):
    cid = lax.axis_index("c")
    sid = lax.axis_index("s")
    row0 = sid * ROWS_PER_TILE
    pltpu.sync_copy(zinit, acc_sh.at[pl.ds(row0, ROWS_PER_TILE)])
    pltpu.sync_copy(ones_rows, rows_v)
    plsc.subcore_barrier()

    def run(d_hbm):
        def outer(g0, carry):
            hs = []
            for b in range(NBUF_C):
                base = sid * EPT + (g0 * NBUF_C + b) * CHUNK
                hs.append(pltpu.async_copy(d_hbm.at[pl.ds(base, CHUNK)],
                                           dst_b[b], sem_i[b]))
            ss = []
            for b in range(NBUF_C):
                hs[b].wait()
                ss.append(pltpu.async_copy(rows_v, acc_sh.at[dst_b[b]],
                                           sem_s[b], add=True))
            for b in range(NBUF_C):
                ss[b].wait()
            return carry
        lax.fori_loop(0, OUTER_C, outer, 0)

    @pl.when(cid == 0)
    def _():
        run(d_ui)

    @pl.when(cid == 1)
    def _():
        run(d_iu)

    plsc.subcore_barrier()

    @pl.when(cid == 0)
    def _():
        pltpu.sync_copy(acc_sh.at[pl.ds(row0, ROWS_PER_TILE)],
                        out_i.at[pl.ds(row0, ROWS_PER_TILE)])

    @pl.when(cid == 1)
    def _():
        pltpu.sync_copy(acc_sh.at[pl.ds(row0, ROWS_PER_TILE)],
                        out_u.at[pl.ds(row0, ROWS_PER_TILE)])


def _sc_count(d_ui, d_iu, ones_rows, zinit):
    mesh = plsc.VectorSubcoreMesh(core_axis_name="c", subcore_axis_name="s",
                                  num_cores=NC, num_subcores=NS)
    f = pl.kernel(
        _sc_count_body,
        out_type=(jax.ShapeDtypeStruct((N_PAD, F), jnp.float32),
                  jax.ShapeDtypeStruct((N_PAD, F), jnp.float32)),
        mesh=mesh,
        scratch_types=[
            [pltpu.VMEM((CHUNK,), jnp.int32) for _ in range(NBUF_C)],
            pltpu.VMEM((CHUNK, F), jnp.float32),
            pltpu.VMEM_SHARED((N_PAD, F), jnp.float32),
            [pltpu.SemaphoreType.DMA for _ in range(NBUF_C)],
            [pltpu.SemaphoreType.DMA for _ in range(NBUF_C)],
        ],
    )
    return f(d_ui, d_iu, ones_rows, zinit)


def _pad_edges(ei):
    src = ei[0].astype(jnp.int32)
    dst = ei[1].astype(jnp.int32)
    pad = E_PAD - src.shape[0]
    src = jnp.concatenate([src, jnp.zeros((pad,), jnp.int32)])
    dst = jnp.concatenate([dst, jnp.full((pad,), N, jnp.int32)])
    return src, dst


def kernel(x_user, x_item, edge_index_u2i, edge_index_i2u,
           W1_l_ui, b1_ui, W1_r_ui, W1_l_iu, b1_iu, W1_r_iu,
           W2_l_ui, b2_ui, W2_r_ui, W2_l_iu, b2_iu, W2_r_iu):
    s_ui, d_ui = _pad_edges(edge_index_u2i)
    s_iu, d_iu = _pad_edges(edge_index_i2u)
    zinit = jnp.zeros((ROWS_PER_TILE, F), jnp.float32)
    ones_rows = jnp.ones((CHUNK, F), jnp.float32)

    cnt_i, cnt_u = _sc_count(d_ui, d_iu, ones_rows, zinit)

    tab_u1, r_user1, tab_i1, r_item1 = _mm(
        x_user, x_item,
        jnp.concatenate([W1_l_ui, W1_r_iu], axis=1),
        jnp.concatenate([W1_l_iu, W1_r_ui], axis=1))
    acc_i1, acc_u1 = _sc_scatter(tab_u1, tab_i1, s_ui, d_ui, s_iu, d_iu, zinit)
    x_user1, x_item1 = _epi(acc_u1, cnt_u, r_user1, b1_iu,
                            acc_i1, cnt_i, r_item1, b1_ui, relu=True)

    tab_u2, r_user2, tab_i2, r_item2 = _mm(
        x_user1, x_item1,
        jnp.concatenate([W2_l_ui, W2_r_iu], axis=1),
        jnp.concatenate([W2_l_iu, W2_r_ui], axis=1))
    acc_i2, acc_u2 = _sc_scatter(tab_u2, tab_i2, s_ui, d_ui, s_iu, d_iu, zinit)
    x_user2, x_item2 = _epi(acc_u2, cnt_u, r_user2, b2_iu,
                            acc_i2, cnt_i, r_item2, b2_ui, relu=False)
    return (x_user2, x_item2)


# speedup vs baseline: 1.0216x; 1.0216x over previous
import jax
import jax.numpy as jnp
from jax import lax
from jax.experimental import pallas as pl
from jax.experimental.pallas import tpu as pltpu
from jax.experimental.pallas import tpu_sc as plsc

N = 10000
F = 128
N_PAD = 10112
NS = 16
NC = 2
CHUNK = 128
CHUNKS_PER_TILE = 80
E_PAD = NS * CHUNKS_PER_TILE * CHUNK
EPT = E_PAD // NS
ROWS_PER_TILE = N_PAD // NS
BM = 400


def _mm_body(xu_ref, xi_ref, wu_ref, wi_ref,
             tabu_ref, ru_ref, tabi_ref, ri_ref):
    zu = jnp.dot(xu_ref[...], wu_ref[...], preferred_element_type=jnp.float32)
    tabu_ref[...] = zu[:, :F]
    ru_ref[...] = zu[:, F:]
    zi = jnp.dot(xi_ref[...], wi_ref[...], preferred_element_type=jnp.float32)
    tabi_ref[...] = zi[:, :F]
    ri_ref[...] = zi[:, F:]


def _mm(x_u, x_i, w_u, w_i):
    k = x_u.shape[1]
    return pl.pallas_call(
        _mm_body,
        grid=(N // BM,),
        in_specs=[
            pl.BlockSpec((BM, k), lambda i: (i, 0)),
            pl.BlockSpec((BM, k), lambda i: (i, 0)),
            pl.BlockSpec((k, 2 * F), lambda i: (0, 0)),
            pl.BlockSpec((k, 2 * F), lambda i: (0, 0)),
        ],
        out_specs=[
            pl.BlockSpec((BM, F), lambda i: (i, 0)),
            pl.BlockSpec((BM, F), lambda i: (i, 0)),
            pl.BlockSpec((BM, F), lambda i: (i, 0)),
            pl.BlockSpec((BM, F), lambda i: (i, 0)),
        ],
        out_shape=[
            jax.ShapeDtypeStruct((N, F), jnp.float32),
            jax.ShapeDtypeStruct((N, F), jnp.float32),
            jax.ShapeDtypeStruct((N, F), jnp.float32),
            jax.ShapeDtypeStruct((N, F), jnp.float32),
        ],
    )(x_u, x_i, w_u, w_i)


def _epi_body_relu(au, cu, ru, bu, ai, ci, ri, bi, ou, oi):
    _epi_common(au, cu, ru, bu, ai, ci, ri, bi, ou, oi, True)


def _epi_body_lin(au, cu, ru, bu, ai, ci, ri, bi, ou, oi):
    _epi_common(au, cu, ru, bu, ai, ci, ri, bi, ou, oi, False)


def _epi_common(au, cu, ru, bu, ai, ci, ri, bi, ou, oi, relu):
    def one(acc_ref, cnt_ref, r_ref, b_ref, o_ref):
        cnt = jnp.maximum(cnt_ref[...], 1.0)
        out = acc_ref[...] / cnt + b_ref[...] + r_ref[...]
        if relu:
            out = jnp.maximum(out, 0.0)
        o_ref[...] = out
    one(au, cu, ru, bu, ou)
    one(ai, ci, ri, bi, oi)


def _epi(acc_u, cnt_u, r_u, b_u, acc_i, cnt_i, r_i, b_i, relu):
    body = _epi_body_relu if relu else _epi_body_lin
    blk = pl.BlockSpec((BM, F), lambda i: (i, 0))
    bias = pl.BlockSpec((1, F), lambda i: (0, 0))
    return pl.pallas_call(
        body,
        grid=(N // BM,),
        in_specs=[blk, blk, blk, bias, blk, blk, blk, bias],
        out_specs=[blk, blk],
        out_shape=[
            jax.ShapeDtypeStruct((N, F), jnp.float32),
            jax.ShapeDtypeStruct((N, F), jnp.float32),
        ],
    )(acc_u, cnt_u, r_u, b_u.reshape(1, F),
      acc_i, cnt_i, r_i, b_i.reshape(1, F))


CH_S = 128
NBUF = 2
KCH = 10
OUTER = EPT // CH_S // KCH
NBUF_C = 10
OUTER_C = CHUNKS_PER_TILE // NBUF_C


def _sc_scatter_body(tab_u, tab_i, s_ui, d_ui, s_iu, d_iu, zinit,
                     out_i, out_u, src_b, dst_b, rows_b, acc_sh,
                     sem_i, sem_g, sem_s):
    cid = lax.axis_index("c")
    sid = lax.axis_index("s")
    row0 = sid * ROWS_PER_TILE
    pltpu.sync_copy(zinit, acc_sh.at[pl.ds(row0, ROWS_PER_TILE)])
    plsc.subcore_barrier()

    def run(tab, s_hbm, d_hbm):
        def outer(g0, carry):
            hs = []
            for j in range(KCH):
                base = sid * EPT + (g0 * KCH + j) * CH_S
                h1 = pltpu.async_copy(s_hbm.at[pl.ds(base, CH_S)],
                                      src_b[j], sem_i[j])
                h2 = pltpu.async_copy(d_hbm.at[pl.ds(base, CH_S)],
                                      dst_b[j], sem_i[j])
                hs.append((h1, h2))
            gs = [None] * KCH
            ss = [None] * KCH
            for j in range(KCH):
                slot = j % NBUF
                if j >= NBUF:
                    ss[j - NBUF].wait()
                hs[j][0].wait()
                gs[j] = pltpu.async_copy(tab.at[src_b[j]],
                                         rows_b[slot], sem_g[slot])
                if j >= 1:
                    gs[j - 1].wait()
                    hs[j - 1][1].wait()
                    ss[j - 1] = pltpu.async_copy(
                        rows_b[(j - 1) % NBUF],
                        acc_sh.at[dst_b[j - 1]], sem_s[(j - 1) % NBUF],
                        add=True)
            gs[KCH - 1].wait()
            hs[KCH - 1][1].wait()
            ss[KCH - 1] = pltpu.async_copy(
                rows_b[(KCH - 1) % NBUF],
                acc_sh.at[dst_b[KCH - 1]], sem_s[(KCH - 1) % NBUF],
                add=True)
            ss[KCH - 2].wait()
            ss[KCH - 1].wait()
            return carry
        lax.fori_loop(0, OUTER, outer, 0)

    @pl.when(cid == 0)
    def _():
        run(tab_u, s_ui, d_ui)

    @pl.when(cid == 1)
    def _():
        run(tab_i, s_iu, d_iu)

    plsc.subcore_barrier()

    @pl.when(cid == 0)
    def _():
        pltpu.sync_copy(acc_sh.at[pl.ds(row0, ROWS_PER_TILE)],
                        out_i.at[pl.ds(row0, ROWS_PER_TILE)])

    @pl.when(cid == 1)
    def _():
        pltpu.sync_copy(acc_sh.at[pl.ds(row0, ROWS_PER_TILE)],
                        out_u.at[pl.ds(row0, ROWS_PER_TILE)])


def _sc_scatter(tab_u, tab_i, s_ui, d_ui, s_iu, d_iu, zinit):
    mesh = plsc.VectorSubcoreMesh(core_axis_name="c", subcore_axis_name="s",
                                  num_cores=NC, num_subcores=NS)
    f = pl.kernel(
        _sc_scatter_body,
        out_type=(jax.ShapeDtypeStruct((N_PAD, F), jnp.float32),
                  jax.ShapeDtypeStruct((N_PAD, F), jnp.float32)),
        mesh=mesh,
        scratch_types=[
            [pltpu.VMEM((CH_S,), jnp.int32) for _ in range(KCH)],
            [pltpu.VMEM((CH_S,), jnp.int32) for _ in range(KCH)],
            [pltpu.VMEM((CH_S, F), jnp.float32) for _ in range(NBUF)],
            pltpu.VMEM_SHARED((N_PAD, F), jnp.float32),
            [pltpu.SemaphoreType.DMA for _ in range(KCH)],
            [pltpu.SemaphoreType.DMA for _ in range(NBUF)],
            [pltpu.SemaphoreType.DMA for _ in range(NBUF)],
        ],
    )
    return f(tab_u, tab_i, s_ui, d_ui, s_iu, d_iu, zinit)


def _sc_count_body(d_ui, d_iu, ones_rows, zinit, out_i, out_u,
                   dst_b, rows_v, acc_sh, sem_i, sem_s):
    cid = lax.axis_index("c")
    sid = lax.axis_index("s")
    row0 = sid * ROWS_PER_TILE
    pltpu.sync_copy(zinit, acc_sh.at[pl.ds(row0, ROWS_PER_TILE)])
    pltpu.sync_copy(ones_rows, rows_v)
    plsc.subcore_barrier()

    def run(d_hbm):
        def outer(g0, carry):
            hs = []
            for b in range(NBUF_C):
                base = sid * EPT + (g0 * NBUF_C + b) * CHUNK
                hs.append(pltpu.async_copy(d_hbm.at[pl.ds(base, CHUNK)],
                                           dst_b[b], sem_i[b]))
            ss = []
            for b in range(NBUF_C):
                hs[b].wait()
                ss.append(pltpu.async_copy(rows_v, acc_sh.at[dst_b[b]],
                                           sem_s[b], add=True))
            for b in range(NBUF_C):
                ss[b].wait()
            return carry
        lax.fori_loop(0, OUTER_C, outer, 0)

    @pl.when(cid == 0)
    def _():
        run(d_ui)

    @pl.when(cid == 1)
    def _():
        run(d_iu)

    plsc.subcore_barrier()

    @pl.when(cid == 0)
    def _():
        pltpu.sync_copy(acc_sh.at[pl.ds(row0, ROWS_PER_TILE)],
                        out_i.at[pl.ds(row0, ROWS_PER_TILE)])

    @pl.when(cid == 1)
    def _():
        pltpu.sync_copy(acc_sh.at[pl.ds(row0, ROWS_PER_TILE)],
                        out_u.at[pl.ds(row0, ROWS_PER_TILE)])


def _sc_count(d_ui, d_iu, ones_rows, zinit):
    mesh = plsc.VectorSubcoreMesh(core_axis_name="c", subcore_axis_name="s",
                                  num_cores=NC, num_subcores=NS)
    f = pl.kernel(
        _sc_count_body,
        out_type=(jax.ShapeDtypeStruct((N_PAD, F), jnp.float32),
                  jax.ShapeDtypeStruct((N_PAD, F), jnp.float32)),
        mesh=mesh,
        scratch_types=[
            [pltpu.VMEM((CHUNK,), jnp.int32) for _ in range(NBUF_C)],
            pltpu.VMEM((CHUNK, F), jnp.float32),
            pltpu.VMEM_SHARED((N_PAD, F), jnp.float32),
            [pltpu.SemaphoreType.DMA for _ in range(NBUF_C)],
            [pltpu.SemaphoreType.DMA for _ in range(NBUF_C)],
        ],
    )
    return f(d_ui, d_iu, ones_rows, zinit)


def _pad_edges(ei):
    src = ei[0].astype(jnp.int32)
    dst = ei[1].astype(jnp.int32)
    pad = E_PAD - src.shape[0]
    src = jnp.concatenate([src, jnp.zeros((pad,), jnp.int32)])
    dst = jnp.concatenate([dst, jnp.full((pad,), N, jnp.int32)])
    return src, dst


def kernel(x_user, x_item, edge_index_u2i, edge_index_i2u,
           W1_l_ui, b1_ui, W1_r_ui, W1_l_iu, b1_iu, W1_r_iu,
           W2_l_ui, b2_ui, W2_r_ui, W2_l_iu, b2_iu, W2_r_iu):
    s_ui, d_ui = _pad_edges(edge_index_u2i)
    s_iu, d_iu = _pad_edges(edge_index_i2u)
    zinit = jnp.zeros((ROWS_PER_TILE, F), jnp.float32)
    ones_rows = jnp.ones((CHUNK, F), jnp.float32)

    cnt_i, cnt_u = _sc_count(d_ui, d_iu, ones_rows, zinit)

    tab_u1, r_user1, tab_i1, r_item1 = _mm(
        x_user, x_item,
        jnp.concatenate([W1_l_ui, W1_r_iu], axis=1),
        jnp.concatenate([W1_l_iu, W1_r_ui], axis=1))
    acc_i1, acc_u1 = _sc_scatter(tab_u1, tab_i1, s_ui, d_ui, s_iu, d_iu, zinit)
    x_user1, x_item1 = _epi(acc_u1, cnt_u, r_user1, b1_iu,
                            acc_i1, cnt_i, r_item1, b1_ui, relu=True)

    tab_u2, r_user2, tab_i2, r_item2 = _mm(
        x_user1, x_item1,
        jnp.concatenate([W2_l_ui, W2_r_iu], axis=1),
        jnp.concatenate([W2_l_iu, W2_r_ui], axis=1))
    acc_i2, acc_u2 = _sc_scatter(tab_u2, tab_i2, s_ui, d_ui, s_iu, d_iu, zinit)
    x_user2, x_item2 = _epi(acc_u2, cnt_u, r_user2, b2_iu,
                            acc_i2, cnt_i, r_item2, b2_ui, relu=False)
    return (x_user2, x_item2)
